# baseline (speedup 1.0000x reference)
import functools

import jax
import jax.numpy as jnp
from jax import lax
from jax.experimental import pallas as pl
from jax.experimental.pallas import tpu as pltpu
from jax.experimental.pallas import tpu_sc as plsc

B = 16384
D_ITEM = 64
D_SMALL = 8
D_USED = 80
D_OUT = 128
EPS = 1e-5

_NC = 2
_NS = 16
_NW = _NC * _NS
_BPW = B // _NW
_CH = 128
_NCH = _BPW // _CH
_L = 16
_CS_CAT = 84 * D_SMALL
_CS_LEN = _CS_CAT + 60 * D_SMALL


def _sc_gather(idx3, item128, catshop_flat):
    mesh = plsc.VectorSubcoreMesh(core_axis_name="c", subcore_axis_name="s")

    @functools.partial(
        pl.kernel,
        mesh=mesh,
        compiler_params=pltpu.CompilerParams(use_tc_tiling_on_sc=False,
                                             needs_layout_passes=False),
        out_type=jax.ShapeDtypeStruct((B, D_OUT), jnp.float32),
        scratch_types=[
            pltpu.VMEM((_NCH, _CH), jnp.int32),
            pltpu.VMEM((_NCH, _CH), jnp.int32),
            pltpu.VMEM((_NCH, _CH), jnp.int32),
            pltpu.VMEM((_BPW, D_OUT), jnp.float32),
            pltpu.VMEM((_CS_LEN,), jnp.float32),
            pltpu.SemaphoreType.DMA,
            pltpu.SemaphoreType.DMA,
            pltpu.SemaphoreType.DMA,
            pltpu.SemaphoreType.DMA,
            pltpu.SemaphoreType.DMA,
            pltpu.SemaphoreType.DMA,
        ],
    )
    def k(idx_h, item_h, cs_h, t_h,
          i1_v, i2_v, i3_v, r_v, tab_v,
          sg0, sg1, sg2, sg3, si, sw):
        wid = lax.axis_index("s") * _NC + lax.axis_index("c")
        base = wid * _BPW
        row0 = wid * _NCH
        idx_cp = [
            pltpu.async_copy(idx_h.at[0, pl.ds(row0, _NCH)], i1_v, si),
            pltpu.async_copy(idx_h.at[1, pl.ds(row0, _NCH)], i2_v, si),
            pltpu.async_copy(idx_h.at[2, pl.ds(row0, _NCH)], i3_v, si),
            pltpu.async_copy(cs_h, tab_v, si),
        ]
        for c in idx_cp:
            c.wait()
        sems = (sg0, sg1, sg2, sg3)
        g1 = [pltpu.async_copy(item_h.at[i1_v.at[j]],
                               r_v.at[pl.ds(j * _CH, _CH)], sems[j])
              for j in range(_NCH)]
        lanes = lax.iota(jnp.int32, _L)
        writes = []
        for j in range(_NCH):
            g1[j].wait()
            for g in range(_CH // _L):
                rows = j * _CH + g * _L + lanes
                b2 = i2_v[j, pl.ds(g * _L, _L)] * D_SMALL
                b3 = i3_v[j, pl.ds(g * _L, _L)] * D_SMALL + _CS_CAT
                for c in range(D_SMALL):
                    v2 = plsc.load_gather(tab_v, [b2 + c])
                    plsc.store_scatter(r_v, [rows, lanes * 0 + (D_ITEM + c)],
                                       v2)
                    v3 = plsc.load_gather(tab_v, [b3 + c])
                    plsc.store_scatter(
                        r_v, [rows, lanes * 0 + (D_ITEM + D_SMALL + c)], v3)
            writes.append(pltpu.async_copy(
                r_v.at[pl.ds(j * _CH, _CH)],
                t_h.at[pl.ds(base + j * _CH, _CH)], sw))
        for w in writes:
            w.wait()

    return k(idx3, item128, catshop_flat)


def _mlp_body(t_r, g0_r, b0_r, w1_r, b1_r, g1_r, b1n_r, w2_r, b2_r,
              g2_r, b2n_r, w3_r, b3_r, o_r):
    n = float(B)

    def affine(x, g, b):
        m = jnp.sum(x, axis=0, keepdims=True) / n
        d = x - m
        v = jnp.sum(d * d, axis=0, keepdims=True) / n
        a = g * lax.rsqrt(v + EPS)
        return a, b - m * a

    def dot(x, w):
        return jnp.dot(x, w, preferred_element_type=jnp.float32)

    t = t_r[...]
    a, c = affine(t, g0_r[...], b0_r[...])
    h = jax.nn.relu(dot(t * a + c, w1_r[...]) + b1_r[...])

    a, c = affine(h, g1_r[...], b1n_r[...])
    h = jax.nn.relu(dot(h * a + c, w2_r[...]) + b2_r[...])

    a, c = affine(h, g2_r[...], b2n_r[...])
    o_r[...] = dot(h * a + c, w3_r[...]) + b3_r[...]


def _pad_feat(v):
    return jnp.pad(v, (0, D_OUT - D_USED)).reshape(1, D_OUT)


def kernel(input, item_table, cat_table, shop_table, bn0_g, bn0_b,
           fc1_w, fc1_b, bn1_g, bn1_b, fc2_w, fc2_b, bn2_g, bn2_b,
           out_w, out_b):
    idx3 = input.astype(jnp.int32).T.reshape(3, _NW * _NCH, _CH)
    catshop = jnp.concatenate([cat_table.reshape(-1), shop_table.reshape(-1)])
    item128 = jnp.pad(item_table, ((0, 0), (0, D_OUT - D_ITEM)))
    t = _sc_gather(idx3, item128, catshop)

    w1p = jnp.pad(fc1_w.T, ((0, D_OUT - D_USED), (0, 0)))

    y = pl.pallas_call(
        _mlp_body,
        out_shape=jax.ShapeDtypeStruct((B, 1), jnp.float32),
    )(t,
      _pad_feat(bn0_g), _pad_feat(bn0_b),
      w1p, fc1_b.reshape(1, -1),
      bn1_g.reshape(1, -1), bn1_b.reshape(1, -1),
      fc2_w.T, fc2_b.reshape(1, -1),
      bn2_g.reshape(1, -1), bn2_b.reshape(1, -1),
      out_w.T, out_b.reshape(1, -1))
    return y[:, 0]


# speedup vs baseline: 1.0090x; 1.0090x over previous
import functools

import jax
import jax.numpy as jnp
from jax import lax
from jax.experimental import pallas as pl
from jax.experimental.pallas import tpu as pltpu
from jax.experimental.pallas import tpu_sc as plsc

B = 16384
D_ITEM = 64
D_SMALL = 8
D_USED = 80
D_OUT = 128
EPS = 1e-5

_NC = 2
_NS = 16
_NW = _NC * _NS
_BPW = B // _NW
_CH = 128
_NCH = _BPW // _CH
_L = 16
_CS_CAT = 84 * D_SMALL
_CS_LEN = _CS_CAT + 60 * D_SMALL


def _sc_gather(idx3, item128, catshop_flat):
    mesh = plsc.VectorSubcoreMesh(core_axis_name="c", subcore_axis_name="s")

    @functools.partial(
        pl.kernel,
        mesh=mesh,
        compiler_params=pltpu.CompilerParams(use_tc_tiling_on_sc=False,
                                             needs_layout_passes=False),
        out_type=jax.ShapeDtypeStruct((B, D_OUT), jnp.float32),
        scratch_types=[
            pltpu.VMEM((_NCH, _CH), jnp.int32),
            pltpu.VMEM((_NCH, _CH), jnp.int32),
            pltpu.VMEM((_NCH, _CH), jnp.int32),
            pltpu.VMEM((_BPW, D_OUT), jnp.float32),
            pltpu.VMEM((_CS_LEN,), jnp.float32),
            pltpu.SemaphoreType.DMA,
            pltpu.SemaphoreType.DMA,
            pltpu.SemaphoreType.DMA,
            pltpu.SemaphoreType.DMA,
            pltpu.SemaphoreType.DMA,
            pltpu.SemaphoreType.DMA,
        ],
    )
    def k(idx_h, item_h, cs_h, t_h,
          i1_v, i2_v, i3_v, r_v, tab_v,
          sg0, sg1, sg2, sg3, si, sw):
        wid = lax.axis_index("s") * _NC + lax.axis_index("c")
        base = wid * _BPW
        row0 = wid * _NCH
        c1 = pltpu.async_copy(idx_h.at[0, pl.ds(row0, _NCH)], i1_v, si)
        rest = [
            pltpu.async_copy(idx_h.at[1, pl.ds(row0, _NCH)], i2_v, sw),
            pltpu.async_copy(idx_h.at[2, pl.ds(row0, _NCH)], i3_v, sw),
            pltpu.async_copy(cs_h, tab_v, sw),
        ]
        c1.wait()
        sems = (sg0, sg1, sg2, sg3)
        g1 = [pltpu.async_copy(item_h.at[i1_v.at[j]],
                               r_v.at[pl.ds(j * _CH, _CH)], sems[j])
              for j in range(_NCH)]
        for c in rest:
            c.wait()
        lanes = lax.iota(jnp.int32, _L)
        writes = []
        for j in range(_NCH):
            g1[j].wait()
            for g in range(_CH // _L):
                rows = j * _CH + g * _L + lanes
                b2 = i2_v[j, pl.ds(g * _L, _L)] * D_SMALL
                b3 = i3_v[j, pl.ds(g * _L, _L)] * D_SMALL + _CS_CAT
                for c in range(D_SMALL):
                    v2 = plsc.load_gather(tab_v, [b2 + c])
                    plsc.store_scatter(r_v, [rows, lanes * 0 + (D_ITEM + c)],
                                       v2)
                    v3 = plsc.load_gather(tab_v, [b3 + c])
                    plsc.store_scatter(
                        r_v, [rows, lanes * 0 + (D_ITEM + D_SMALL + c)], v3)
            writes.append(pltpu.async_copy(
                r_v.at[pl.ds(j * _CH, _CH)],
                t_h.at[pl.ds(base + j * _CH, _CH)], sw))
        for w in writes:
            w.wait()

    return k(idx3, item128, catshop_flat)


def _mlp_body(t_r, g0_r, b0_r, w1_r, b1_r, g1_r, b1n_r, w2_r, b2_r,
              g2_r, b2n_r, w3_r, b3_r, o_r):
    n = float(B)

    def affine(x, g, b):
        m = jnp.sum(x, axis=0, keepdims=True) / n
        d = x - m
        v = jnp.sum(d * d, axis=0, keepdims=True) / n
        a = g * lax.rsqrt(v + EPS)
        return a, b - m * a

    def dot(x, w):
        return jnp.dot(x, w, preferred_element_type=jnp.float32)

    t = t_r[...]
    a, c = affine(t, g0_r[...], b0_r[...])
    h = jax.nn.relu(dot(t * a + c, w1_r[...]) + b1_r[...])

    a, c = affine(h, g1_r[...], b1n_r[...])
    h = jax.nn.relu(dot(h * a + c, w2_r[...]) + b2_r[...])

    a, c = affine(h, g2_r[...], b2n_r[...])
    o_r[...] = dot(h * a + c, w3_r[...]) + b3_r[...]


def _pad_feat(v):
    return jnp.pad(v, (0, D_OUT - D_USED)).reshape(1, D_OUT)


def kernel(input, item_table, cat_table, shop_table, bn0_g, bn0_b,
           fc1_w, fc1_b, bn1_g, bn1_b, fc2_w, fc2_b, bn2_g, bn2_b,
           out_w, out_b):
    idx3 = input.astype(jnp.int32).T.reshape(3, _NW * _NCH, _CH)
    catshop = jnp.concatenate([cat_table.reshape(-1), shop_table.reshape(-1)])
    item128 = jnp.pad(item_table, ((0, 0), (0, D_OUT - D_ITEM)))
    t = _sc_gather(idx3, item128, catshop)

    w1p = jnp.pad(fc1_w.T, ((0, D_OUT - D_USED), (0, 0)))

    y = pl.pallas_call(
        _mlp_body,
        out_shape=jax.ShapeDtypeStruct((B, 1), jnp.float32),
    )(t,
      _pad_feat(bn0_g), _pad_feat(bn0_b),
      w1p, fc1_b.reshape(1, -1),
      bn1_g.reshape(1, -1), bn1_b.reshape(1, -1),
      fc2_w.T, fc2_b.reshape(1, -1),
      bn2_g.reshape(1, -1), bn2_b.reshape(1, -1),
      out_w.T, out_b.reshape(1, -1))
    return y[:, 0]
